# baseline (speedup 1.0000x reference)
import functools

import jax
import jax.numpy as jnp
from jax import lax
from jax.experimental import pallas as pl
from jax.experimental.pallas import tpu as pltpu
from jax.experimental.pallas import tpu_sc as plsc

_F32 = jnp.float32
_NW = 32
_CHUNK = 80
_NBUF = 4


def _sc_mesh():
    return plsc.VectorSubcoreMesh(core_axis_name="c", subcore_axis_name="s")


def _make_deg(N, e_pt):
    @functools.partial(
        pl.kernel,
        out_type=jax.ShapeDtypeStruct((2, N), _F32),
        mesh=_sc_mesh(),
        scratch_types=[
            pltpu.VMEM((e_pt,), jnp.int32),
            pltpu.VMEM((1, N), _F32),
            pltpu.VMEM((1,), jnp.int32),
            pltpu.VMEM_SHARED((1, N), _F32),
        ],
        compiler_params=pltpu.CompilerParams(
            use_tc_tiling_on_sc=False, needs_layout_passes=False),
    )
    def deg_kernel(dst_hbm, zeros_hbm, zi_hbm, out_hbm, dst_v, hist, zidx, acc):
        cid = lax.axis_index("c")
        sid = lax.axis_index("s")
        wid = sid * 2 + cid
        pltpu.sync_copy(dst_hbm.at[wid], dst_v)
        pltpu.sync_copy(zi_hbm, zidx)

        @pl.when(sid == 0)
        def _():
            pltpu.sync_copy(zeros_hbm, acc)

        zv = jnp.zeros((16,), _F32)

        @pl.loop(0, N // 16)
        def _(r):
            hist[0, pl.ds(r * 16, 16)] = zv

        ones = jnp.ones((16,), _F32)

        @pl.loop(0, e_pt // 16)
        def _(j):
            idxv = dst_v[pl.ds(j * 16, 16)]
            plsc.addupdate_scatter(hist.at[0], [idxv], ones)

        plsc.subcore_barrier()
        pltpu.sync_copy(hist, acc.at[zidx], add=True)
        plsc.subcore_barrier()

        @pl.when(sid == 0)
        def _():
            pltpu.sync_copy(acc.at[0], out_hbm.at[cid])

    return deg_kernel


def _make_hop(N, H, nchunk):
    rows_pt = N // 16

    @functools.partial(
        pl.kernel,
        out_type=jax.ShapeDtypeStruct((2, N, H), _F32),
        mesh=_sc_mesh(),
        scratch_types=[
            pltpu.VMEM((_NBUF, 2, _CHUNK), jnp.int32),
        ]
        + [pltpu.VMEM((_CHUNK, H), _F32) for _ in range(_NBUF)]
        + [
            pltpu.VMEM_SHARED((N, H), _F32),
        ]
        + [pltpu.SemaphoreType.DMA for _ in range(2 * _NBUF + 1)],
    )
    def hop_kernel(ei_hbm, cs_hbm, zeros_hbm, out_hbm, idx_v, *rest):
        rows = rest[:_NBUF]
        acc = rest[_NBUF]
        gsems = rest[_NBUF + 1:2 * _NBUF + 1]
        isems = rest[2 * _NBUF + 1:3 * _NBUF + 1]
        zsem = rest[3 * _NBUF + 1]
        cid = lax.axis_index("c")
        sid = lax.axis_index("s")
        wid = sid * 2 + cid
        zdesc = pltpu.async_copy(
            zeros_hbm, acc.at[pl.ds(sid * rows_pt, rows_pt)], zsem)

        def start(j, b):
            pltpu.async_copy(ei_hbm.at[wid, j], idx_v.at[b], isems[b])

        def wait_idx_and_gather(j, b):
            pltpu.make_async_copy(ei_hbm.at[wid, j], idx_v.at[b], isems[b]).wait()
            pltpu.async_copy(cs_hbm.at[idx_v.at[b, 0]], rows[b], gsems[b])

        for b in range(_NBUF):
            start(b, b)
        for b in range(_NBUF):
            wait_idx_and_gather(b, b)
        zdesc.wait()
        plsc.subcore_barrier()

        niter = -(-nchunk // _NBUF) * _NBUF

        @pl.loop(0, niter, step=_NBUF)
        def _(i):
            for b in range(_NBUF):
                j = i + b

                @pl.when(j < nchunk)
                def _():
                    pltpu.make_async_copy(
                        cs_hbm.at[idx_v.at[b, 0]], rows[b], gsems[b]).wait()
                    pltpu.sync_copy(rows[b], acc.at[idx_v.at[b, 1]], add=True)

                @pl.when(j + _NBUF < nchunk)
                def _():
                    start(j + _NBUF, b)
                    wait_idx_and_gather(j + _NBUF, b)

        plsc.subcore_barrier()
        pltpu.sync_copy(
            acc.at[pl.ds(sid * rows_pt, rows_pt)],
            out_hbm.at[cid, pl.ds(sid * rows_pt, rows_pt)],
        )

    return hop_kernel


def _combine(parts, norm2d, rows_blk=1000):
    _, N, H = parts.shape

    def body(p_ref, n_ref, f_ref, s_ref):
        p = p_ref[...]
        nv = n_ref[...]
        f = (p[0] + p[1]) * nv
        f_ref[...] = f
        s_ref[...] = f * nv

    return pl.pallas_call(
        body,
        grid=(N // rows_blk,),
        in_specs=[
            pl.BlockSpec((2, rows_blk, H), lambda i: (0, i, 0)),
            pl.BlockSpec((rows_blk, 1), lambda i: (i, 0)),
        ],
        out_specs=[
            pl.BlockSpec((rows_blk, H), lambda i: (i, 0)),
            pl.BlockSpec((rows_blk, H), lambda i: (i, 0)),
        ],
        out_shape=[jax.ShapeDtypeStruct((N, H), _F32)] * 2,
    )(parts, norm2d)


def _mm_relu(feats, W, b, rows_blk=1000):
    N, H = feats[0].shape
    nf = len(feats)

    def body(*refs):
        f_refs = refs[:nf]
        w_ref, b_ref, o_ref = refs[nf], refs[nf + 1], refs[nf + 2]
        w = w_ref[...]
        acc = b_ref[...].astype(_F32)
        for k in range(nf):
            acc = acc + jnp.dot(
                f_refs[k][...],
                w[k * H:(k + 1) * H, :],
                precision=lax.Precision.HIGHEST,
                preferred_element_type=_F32,
            )
        o_ref[...] = jnp.maximum(acc, 0.0)

    return pl.pallas_call(
        body,
        grid=(N // rows_blk,),
        in_specs=[pl.BlockSpec((rows_blk, H), lambda i: (i, 0)) for _ in range(nf)]
        + [
            pl.BlockSpec(W.shape, lambda i: (0, 0)),
            pl.BlockSpec((1, W.shape[1]), lambda i: (0, 0)),
        ],
        out_specs=pl.BlockSpec((rows_blk, W.shape[1]), lambda i: (i, 0)),
        out_shape=jax.ShapeDtypeStruct((N, W.shape[1]), _F32),
    )(*feats, W, b.reshape(1, -1))


def kernel(x, edge_index, W1, b1, W2, b2):
    N, D = x.shape
    H = W1.shape[1]
    E = edge_index.shape[1]
    nchunk = E // (_NW * _CHUNK)
    assert E == _NW * nchunk * _CHUNK

    Np = -(-N // 128) * 128
    blk = Np // 8
    x = jnp.pad(x, ((0, Np - N), (0, 0)))

    ei3 = jnp.stack(
        (edge_index[0].reshape(_NW, nchunk, _CHUNK),
         edge_index[1].reshape(_NW, nchunk, _CHUNK)), axis=2)
    zeros_h = jnp.zeros((Np // 16, H), _F32)
    e_pt = E // _NW
    dst2 = edge_index[1].reshape(_NW, e_pt)
    zeros_n = jnp.zeros((1, Np), _F32)
    zi = jnp.zeros((1,), jnp.int32)

    deg_parts = _make_deg(Np, e_pt)(dst2, zeros_n, zi)
    deg = deg_parts[0] + deg_parts[1]
    norm = jnp.clip(deg, 1.0) ** -0.5
    norm2d = norm[:, None]

    hop = _make_hop(Np, H, nchunk)
    h = x
    for (W, b) in ((W1, b1), (W2, b2)):
        s = h * norm2d
        feats = [h]
        for _ in range(3):
            parts = hop(ei3, s, zeros_h)
            f, s = _combine(parts, norm2d, rows_blk=blk)
            feats.append(f)
        h = _mm_relu(feats, W, b, rows_blk=blk)
    return h[:N]


# speedup vs baseline: 1.0810x; 1.0810x over previous
import functools

import jax
import jax.numpy as jnp
from jax import lax
from jax.experimental import pallas as pl
from jax.experimental.pallas import tpu as pltpu
from jax.experimental.pallas import tpu_sc as plsc

_F32 = jnp.float32
_NW = 32
_CHUNK = 100
_NBUF = 3


def _sc_mesh():
    return plsc.VectorSubcoreMesh(core_axis_name="c", subcore_axis_name="s")


def _make_deg(N, e_pt):
    @functools.partial(
        pl.kernel,
        out_type=jax.ShapeDtypeStruct((2, N), _F32),
        mesh=_sc_mesh(),
        scratch_types=[
            pltpu.VMEM((e_pt,), jnp.int32),
            pltpu.VMEM((1, N), _F32),
            pltpu.VMEM((1,), jnp.int32),
            pltpu.VMEM_SHARED((1, N), _F32),
        ],
        compiler_params=pltpu.CompilerParams(
            use_tc_tiling_on_sc=False, needs_layout_passes=False),
    )
    def deg_kernel(dst_hbm, zeros_hbm, zi_hbm, out_hbm, dst_v, hist, zidx, acc):
        cid = lax.axis_index("c")
        sid = lax.axis_index("s")
        wid = sid * 2 + cid
        pltpu.sync_copy(dst_hbm.at[wid], dst_v)
        pltpu.sync_copy(zi_hbm, zidx)

        @pl.when(sid == 0)
        def _():
            pltpu.sync_copy(zeros_hbm, acc)

        zv = jnp.zeros((16,), _F32)

        @pl.loop(0, N // 16)
        def _(r):
            hist[0, pl.ds(r * 16, 16)] = zv

        ones = jnp.ones((16,), _F32)

        @pl.loop(0, e_pt // 16)
        def _(j):
            idxv = dst_v[pl.ds(j * 16, 16)]
            plsc.addupdate_scatter(hist.at[0], [idxv], ones)

        plsc.subcore_barrier()
        pltpu.sync_copy(hist, acc.at[zidx], add=True)
        plsc.subcore_barrier()

        @pl.when(sid == 0)
        def _():
            pltpu.sync_copy(acc.at[0], out_hbm.at[cid])

    return deg_kernel


def _make_hop(N, H, nchunk):
    rows_pt = N // 16

    @functools.partial(
        pl.kernel,
        out_type=jax.ShapeDtypeStruct((2, N, H), _F32),
        mesh=_sc_mesh(),
        scratch_types=[
            pltpu.VMEM((_NBUF, 2, _CHUNK), jnp.int32),
        ]
        + [pltpu.VMEM((_CHUNK, H), _F32) for _ in range(_NBUF)]
        + [
            pltpu.VMEM_SHARED((N, H), _F32),
        ]
        + [pltpu.SemaphoreType.DMA for _ in range(2 * _NBUF + 1)],
    )
    def hop_kernel(ei_hbm, cs_hbm, zeros_hbm, out_hbm, idx_v, *rest):
        rows = rest[:_NBUF]
        acc = rest[_NBUF]
        gsems = rest[_NBUF + 1:2 * _NBUF + 1]
        isems = rest[2 * _NBUF + 1:3 * _NBUF + 1]
        zsem = rest[3 * _NBUF + 1]
        cid = lax.axis_index("c")
        sid = lax.axis_index("s")
        wid = sid * 2 + cid
        zdesc = pltpu.async_copy(
            zeros_hbm, acc.at[pl.ds(sid * rows_pt, rows_pt)], zsem)

        def start(j, b):
            pltpu.async_copy(ei_hbm.at[wid, j], idx_v.at[b], isems[b])

        def wait_idx_and_gather(j, b):
            pltpu.make_async_copy(ei_hbm.at[wid, j], idx_v.at[b], isems[b]).wait()
            pltpu.async_copy(cs_hbm.at[idx_v.at[b, 0]], rows[b], gsems[b])

        for b in range(_NBUF):
            start(b, b)
        for b in range(_NBUF):
            wait_idx_and_gather(b, b)
        zdesc.wait()
        plsc.subcore_barrier()

        niter = -(-nchunk // _NBUF) * _NBUF

        @pl.loop(0, niter, step=_NBUF)
        def _(i):
            for b in range(_NBUF):
                j = i + b

                @pl.when(j < nchunk)
                def _():
                    pltpu.make_async_copy(
                        cs_hbm.at[idx_v.at[b, 0]], rows[b], gsems[b]).wait()
                    pltpu.sync_copy(rows[b], acc.at[idx_v.at[b, 1]], add=True)

                @pl.when(j + _NBUF < nchunk)
                def _():
                    start(j + _NBUF, b)
                    wait_idx_and_gather(j + _NBUF, b)

        plsc.subcore_barrier()
        pltpu.sync_copy(
            acc.at[pl.ds(sid * rows_pt, rows_pt)],
            out_hbm.at[cid, pl.ds(sid * rows_pt, rows_pt)],
        )

    return hop_kernel


def _combine(parts, norm2d, rows_blk=1000):
    _, N, H = parts.shape

    def body(p_ref, n_ref, f_ref, s_ref):
        p = p_ref[...]
        nv = n_ref[...]
        f = (p[0] + p[1]) * nv
        f_ref[...] = f
        s_ref[...] = f * nv

    return pl.pallas_call(
        body,
        grid=(N // rows_blk,),
        in_specs=[
            pl.BlockSpec((2, rows_blk, H), lambda i: (0, i, 0)),
            pl.BlockSpec((rows_blk, 1), lambda i: (i, 0)),
        ],
        out_specs=[
            pl.BlockSpec((rows_blk, H), lambda i: (i, 0)),
            pl.BlockSpec((rows_blk, H), lambda i: (i, 0)),
        ],
        out_shape=[jax.ShapeDtypeStruct((N, H), _F32)] * 2,
    )(parts, norm2d)


def _mm_relu(feats, W, b, rows_blk=1000):
    N, H = feats[0].shape
    nf = len(feats)

    def body(*refs):
        f_refs = refs[:nf]
        w_ref, b_ref, o_ref = refs[nf], refs[nf + 1], refs[nf + 2]
        w = w_ref[...]
        acc = b_ref[...].astype(_F32)
        for k in range(nf):
            acc = acc + jnp.dot(
                f_refs[k][...],
                w[k * H:(k + 1) * H, :],
                precision=lax.Precision.HIGHEST,
                preferred_element_type=_F32,
            )
        o_ref[...] = jnp.maximum(acc, 0.0)

    return pl.pallas_call(
        body,
        grid=(N // rows_blk,),
        in_specs=[pl.BlockSpec((rows_blk, H), lambda i: (i, 0)) for _ in range(nf)]
        + [
            pl.BlockSpec(W.shape, lambda i: (0, 0)),
            pl.BlockSpec((1, W.shape[1]), lambda i: (0, 0)),
        ],
        out_specs=pl.BlockSpec((rows_blk, W.shape[1]), lambda i: (i, 0)),
        out_shape=jax.ShapeDtypeStruct((N, W.shape[1]), _F32),
    )(*feats, W, b.reshape(1, -1))


def kernel(x, edge_index, W1, b1, W2, b2):
    N, D = x.shape
    H = W1.shape[1]
    E = edge_index.shape[1]
    nchunk = E // (_NW * _CHUNK)
    assert E == _NW * nchunk * _CHUNK

    Np = -(-N // 128) * 128
    blk = Np // 8
    x = jnp.pad(x, ((0, Np - N), (0, 0)))

    ei3 = jnp.stack(
        (edge_index[0].reshape(_NW, nchunk, _CHUNK),
         edge_index[1].reshape(_NW, nchunk, _CHUNK)), axis=2)
    zeros_h = jnp.zeros((Np // 16, H), _F32)
    e_pt = E // _NW
    dst2 = edge_index[1].reshape(_NW, e_pt)
    zeros_n = jnp.zeros((1, Np), _F32)
    zi = jnp.zeros((1,), jnp.int32)

    deg_parts = _make_deg(Np, e_pt)(dst2, zeros_n, zi)
    deg = deg_parts[0] + deg_parts[1]
    norm = jnp.clip(deg, 1.0) ** -0.5
    norm2d = norm[:, None]

    hop = _make_hop(Np, H, nchunk)
    h = x
    for (W, b) in ((W1, b1), (W2, b2)):
        s = h * norm2d
        feats = [h]
        for _ in range(3):
            parts = hop(ei3, s, zeros_h)
            f, s = _combine(parts, norm2d, rows_blk=blk)
            feats.append(f)
        h = _mm_relu(feats, W, b, rows_blk=blk)
    return h[:N]


# speedup vs baseline: 1.1776x; 1.0893x over previous
import functools

import jax
import jax.numpy as jnp
from jax import lax
from jax.experimental import pallas as pl
from jax.experimental.pallas import tpu as pltpu
from jax.experimental.pallas import tpu_sc as plsc

_F32 = jnp.float32
_NW = 32
_CHUNK = 100
_NBUF = 3


def _sc_mesh():
    return plsc.VectorSubcoreMesh(core_axis_name="c", subcore_axis_name="s")


def _make_deg(N, e_pt):
    @functools.partial(
        pl.kernel,
        out_type=jax.ShapeDtypeStruct((2, N), _F32),
        mesh=_sc_mesh(),
        scratch_types=[
            pltpu.VMEM((e_pt,), jnp.int32),
            pltpu.VMEM((1, N), _F32),
            pltpu.VMEM((1,), jnp.int32),
            pltpu.VMEM_SHARED((1, N), _F32),
        ],
        compiler_params=pltpu.CompilerParams(
            use_tc_tiling_on_sc=False, needs_layout_passes=False),
    )
    def deg_kernel(dst_hbm, zeros_hbm, zi_hbm, out_hbm, dst_v, hist, zidx, acc):
        cid = lax.axis_index("c")
        sid = lax.axis_index("s")
        wid = sid * 2 + cid
        pltpu.sync_copy(dst_hbm.at[wid], dst_v)
        pltpu.sync_copy(zi_hbm, zidx)

        @pl.when(sid == 0)
        def _():
            pltpu.sync_copy(zeros_hbm, acc)

        zv = jnp.zeros((16,), _F32)

        @pl.loop(0, N // 16)
        def _(r):
            hist[0, pl.ds(r * 16, 16)] = zv

        ones = jnp.ones((16,), _F32)

        @pl.loop(0, e_pt // 16)
        def _(j):
            idxv = dst_v[pl.ds(j * 16, 16)]
            plsc.addupdate_scatter(hist.at[0], [idxv], ones)

        plsc.subcore_barrier()
        pltpu.sync_copy(hist, acc.at[zidx], add=True)
        plsc.subcore_barrier()

        @pl.when(sid == 0)
        def _():
            pltpu.sync_copy(acc.at[0], out_hbm.at[cid])

    return deg_kernel


def _make_hop(N, H, nchunk):
    rows_pt = N // 16

    @functools.partial(
        pl.kernel,
        out_type=jax.ShapeDtypeStruct((2, N, H), _F32),
        mesh=_sc_mesh(),
        scratch_types=[
            pltpu.VMEM((_NBUF, 2, _CHUNK), jnp.int32),
        ]
        + [pltpu.VMEM((_CHUNK, H), _F32) for _ in range(_NBUF)]
        + [
            pltpu.VMEM_SHARED((N, H), _F32),
        ]
        + [pltpu.SemaphoreType.DMA for _ in range(2 * _NBUF + 2)],
    )
    def hop_kernel(ei_hbm, cs_hbm, zeros_hbm, out_hbm, idx_v, *rest):
        rows = rest[:_NBUF]
        acc = rest[_NBUF]
        gsems = rest[_NBUF + 1:2 * _NBUF + 1]
        isems = rest[2 * _NBUF + 1:3 * _NBUF + 1]
        zsem = rest[3 * _NBUF + 1]
        ssem = rest[3 * _NBUF + 2]
        cid = lax.axis_index("c")
        sid = lax.axis_index("s")
        wid = sid * 2 + cid
        zdesc = pltpu.async_copy(
            zeros_hbm, acc.at[pl.ds(sid * rows_pt, rows_pt)], zsem)

        def start(j, b):
            pltpu.async_copy(ei_hbm.at[wid, j], idx_v.at[b], isems[b])

        def wait_idx_and_gather(j, b):
            pltpu.make_async_copy(ei_hbm.at[wid, j], idx_v.at[b], isems[b]).wait()
            pltpu.async_copy(cs_hbm.at[idx_v.at[b, 0]], rows[b], gsems[b])

        def wait_scatter(b):
            pltpu.make_async_copy(rows[b], acc.at[idx_v.at[b, 1]], ssem).wait()

        for b in range(_NBUF):
            start(b, b)
        for b in range(_NBUF):
            wait_idx_and_gather(b, b)
        zdesc.wait()
        plsc.subcore_barrier()

        niter = -(-nchunk // _NBUF) * _NBUF

        @pl.loop(0, niter, step=_NBUF)
        def _(i):
            for b in range(_NBUF):
                j = i + b
                pb = (b - 1) % _NBUF

                @pl.when(j < nchunk)
                def _():
                    pltpu.make_async_copy(
                        cs_hbm.at[idx_v.at[b, 0]], rows[b], gsems[b]).wait()

                    @pl.when(j > 0)
                    def _():
                        wait_scatter(b)

                    pltpu.async_copy(
                        rows[b], acc.at[idx_v.at[b, 1]], ssem, add=True)

                jp = j + _NBUF - 1

                @pl.when(jnp.logical_and(j > 0, jp < nchunk))
                def _():
                    start(jp, pb)
                    wait_idx_and_gather(jp, pb)

        wait_scatter(0)
        plsc.subcore_barrier()
        pltpu.sync_copy(
            acc.at[pl.ds(sid * rows_pt, rows_pt)],
            out_hbm.at[cid, pl.ds(sid * rows_pt, rows_pt)],
        )

    return hop_kernel


def _combine(parts, norm2d, rows_blk=1000):
    _, N, H = parts.shape

    def body(p_ref, n_ref, f_ref, s_ref):
        p = p_ref[...]
        nv = n_ref[...]
        f = (p[0] + p[1]) * nv
        f_ref[...] = f
        s_ref[...] = f * nv

    return pl.pallas_call(
        body,
        grid=(N // rows_blk,),
        in_specs=[
            pl.BlockSpec((2, rows_blk, H), lambda i: (0, i, 0)),
            pl.BlockSpec((rows_blk, 1), lambda i: (i, 0)),
        ],
        out_specs=[
            pl.BlockSpec((rows_blk, H), lambda i: (i, 0)),
            pl.BlockSpec((rows_blk, H), lambda i: (i, 0)),
        ],
        out_shape=[jax.ShapeDtypeStruct((N, H), _F32)] * 2,
    )(parts, norm2d)


def _mm_relu(feats, W, b, rows_blk=1000):
    N, H = feats[0].shape
    nf = len(feats)

    def body(*refs):
        f_refs = refs[:nf]
        w_ref, b_ref, o_ref = refs[nf], refs[nf + 1], refs[nf + 2]
        w = w_ref[...]
        acc = b_ref[...].astype(_F32)
        for k in range(nf):
            acc = acc + jnp.dot(
                f_refs[k][...],
                w[k * H:(k + 1) * H, :],
                precision=lax.Precision.HIGHEST,
                preferred_element_type=_F32,
            )
        o_ref[...] = jnp.maximum(acc, 0.0)

    return pl.pallas_call(
        body,
        grid=(N // rows_blk,),
        in_specs=[pl.BlockSpec((rows_blk, H), lambda i: (i, 0)) for _ in range(nf)]
        + [
            pl.BlockSpec(W.shape, lambda i: (0, 0)),
            pl.BlockSpec((1, W.shape[1]), lambda i: (0, 0)),
        ],
        out_specs=pl.BlockSpec((rows_blk, W.shape[1]), lambda i: (i, 0)),
        out_shape=jax.ShapeDtypeStruct((N, W.shape[1]), _F32),
    )(*feats, W, b.reshape(1, -1))


def kernel(x, edge_index, W1, b1, W2, b2):
    N, D = x.shape
    H = W1.shape[1]
    E = edge_index.shape[1]
    nchunk = E // (_NW * _CHUNK)
    assert E == _NW * nchunk * _CHUNK

    Np = -(-N // 128) * 128
    blk = Np // 8
    x = jnp.pad(x, ((0, Np - N), (0, 0)))

    ei3 = jnp.stack(
        (edge_index[0].reshape(_NW, nchunk, _CHUNK),
         edge_index[1].reshape(_NW, nchunk, _CHUNK)), axis=2)
    zeros_h = jnp.zeros((Np // 16, H), _F32)
    e_pt = E // _NW
    dst2 = edge_index[1].reshape(_NW, e_pt)
    zeros_n = jnp.zeros((1, Np), _F32)
    zi = jnp.zeros((1,), jnp.int32)

    deg_parts = _make_deg(Np, e_pt)(dst2, zeros_n, zi)
    deg = deg_parts[0] + deg_parts[1]
    norm = jnp.clip(deg, 1.0) ** -0.5
    norm2d = norm[:, None]

    hop = _make_hop(Np, H, nchunk)
    h = x
    for (W, b) in ((W1, b1), (W2, b2)):
        s = h * norm2d
        feats = [h]
        for _ in range(3):
            parts = hop(ei3, s, zeros_h)
            f, s = _combine(parts, norm2d, rows_blk=blk)
            feats.append(f)
        h = _mm_relu(feats, W, b, rows_blk=blk)
    return h[:N]
